# baseline (speedup 1.0000x reference)
import functools

import jax
import jax.numpy as jnp
from jax import lax
from jax.experimental import pallas as pl
from jax.experimental.pallas import tpu as pltpu
from jax.experimental.pallas import tpu_sc as plsc

N1 = 16384
N2 = 4096
C = 512
K = 5
BLK = 256
NBLK = N1 // BLK
BLK2 = 2048
EPS = 1e-5
FINF = 3.0e38

NW = 32
QW = N1 // NW
CH = 8
NCH = QW // CH
KCH = CH * K
NJ = C // 16


def _lin1_body(x1_ref, w1t_ref, b1_ref, y1_ref, stats_ref):
    i = pl.program_id(0)
    y1 = jnp.dot(x1_ref[...], w1t_ref[...],
                 preferred_element_type=jnp.float32) + b1_ref[...]
    y1_ref[...] = y1

    @pl.when(i == 0)
    def _():
        stats_ref[...] = jnp.zeros_like(stats_ref)

    s1 = jnp.sum(y1, axis=0)
    s2 = jnp.sum(y1 * y1, axis=0)
    stats_ref[...] += jnp.stack([s1, s2])


def _knn_body(p1_ref, p2t_ref, idx_ref):
    p1b = p1_ref[...]
    p2t = p2t_ref[...]
    rn1 = jnp.sum(p1b * p1b, axis=1, keepdims=True)
    rn2 = jnp.sum(p2t * p2t, axis=0, keepdims=True)
    pp = jnp.dot(p1b.astype(jnp.bfloat16), p2t.astype(jnp.bfloat16),
                 preferred_element_type=jnp.float32)
    d2 = rn1 - 2.0 * pp + rn2

    iota = lax.broadcasted_iota(jnp.int32, (BLK, N2), 1)
    work = d2
    cols = []
    for k in range(K):
        m = jnp.min(work, axis=1, keepdims=True)
        sel = work <= m
        idxk = jnp.min(jnp.where(sel, iota, N2), axis=1, keepdims=True)
        cols.append(idxk)
        if k < K - 1:
            work = jnp.where(iota == idxk, FINF, work)
    idx_ref[...] = jnp.concatenate(cols, axis=1)


def _interp_body(idx_hbm, p1pad_hbm, p2pad_hbm, x2_hbm, out_hbm,
                 idx_v, p1loc_v, p2loc_v, frows0_v, frows1_v,
                 out0_v, out1_v, semf0, semf1, semo0, semo1):
    cid = lax.axis_index("c")
    sid = lax.axis_index("s")
    wid = sid * 2 + cid
    qbase = wid * QW

    frows_b = (frows0_v, frows1_v)
    out_b = (out0_v, out1_v)
    semf_b = (semf0, semf1)
    semo_b = (semo0, semo1)

    pltpu.sync_copy(idx_hbm.at[pl.ds(qbase * K, QW * K + 16)], idx_v)
    pltpu.sync_copy(p1pad_hbm.at[pl.ds(qbase * 16, QW * 16)], p1loc_v)
    pltpu.sync_copy(p2pad_hbm, p2loc_v)

    def issue(ch, b):
        pltpu.async_copy(x2_hbm.at[idx_v.at[pl.ds(ch * KCH, KCH)]],
                         frows_b[b], semf_b[b])

    issue(0, 0)
    issue(1, 1)

    @pl.loop(0, NCH, step=2)
    def _chunk_pair(i):
        for b in range(2):
            ch = i + b
            qoff = ch * CH
            pltpu.make_async_copy(x2_hbm.at[idx_v.at[pl.ds(0, KCH)]],
                                  frows_b[b], semf_b[b]).wait()
            frows_v = frows_b[b]
            out_v = out_b[b]

            @plsc.parallel_loop(0, CH, unroll=4)
            def q_body(q):
                pq = p1loc_v[pl.ds((qoff + q) * 16, 16)]
                iv = idx_v[pl.ds((qoff + q) * K, 16)]
                wks = []
                for k in range(K):
                    ck = p2loc_v[pl.ds(iv[k] * 16, 16)]
                    d = ck - pq
                    sq = d * d
                    dk = sq[0] + sq[1] + sq[2]
                    wk = 1.0 / jnp.maximum(jnp.full((16,), dk, jnp.float32),
                                           1e-10)
                    wks.append(wk)
                winv = 1.0 / (wks[0] + wks[1] + wks[2] + wks[3] + wks[4])
                wn = [wk * winv for wk in wks]
                r = q * K
                for j in range(NJ):
                    sl = pl.ds(j * 16, 16)
                    acc = (wn[0] * frows_v[r, sl]
                           + wn[1] * frows_v[r + 1, sl]
                           + wn[2] * frows_v[r + 2, sl]
                           + wn[3] * frows_v[r + 3, sl]
                           + wn[4] * frows_v[r + 4, sl])
                    out_v[q, sl] = acc

            pltpu.sync_copy(out_v, out_hbm.at[pl.ds(qbase + qoff, CH), :])

            @pl.when(ch + 2 < NCH)
            def _():
                issue(ch + 2, b)


def _fin_body(y1_ref, interp_ref, stats_ref, g1_ref, be1_ref,
              w2t_ref, b2_ref, out_ref):
    h2 = jnp.dot(interp_ref[...], w2t_ref[...],
                 preferred_element_type=jnp.float32) + b2_ref[...]
    h2 = jnp.maximum(h2, 0.0)
    stats = stats_ref[...]
    mean = stats[0:1, :] * (1.0 / N1)
    var = stats[1:2, :] * (1.0 / N1) - mean * mean
    scale = g1_ref[...] * lax.rsqrt(var + EPS)
    h1 = jnp.maximum((y1_ref[...] - mean) * scale + be1_ref[...], 0.0)
    out_ref[...] = h1 + h2


def kernel(p1, x1, o1, p2, x2, o2, W1, b1, g1, be1, W2, b2):
    p2t = p2.T
    w1t = W1.T
    w2t = W2.T
    b1r = b1.reshape(1, C)
    b2r = b2.reshape(1, C)
    g1r = g1.reshape(1, C)
    be1r = be1.reshape(1, C)
    p1pad = jnp.pad(p1, ((0, 0), (0, 13))).reshape(N1 * 16)
    p2pad = jnp.pad(p2, ((0, 0), (0, 13))).reshape(N2 * 16)

    idx = pl.pallas_call(
        _knn_body,
        grid=(NBLK,),
        in_specs=[
            pl.BlockSpec((BLK, 3), lambda i: (i, 0)),
            pl.BlockSpec((3, N2), lambda i: (0, 0)),
        ],
        out_specs=pl.BlockSpec((BLK, K), lambda i: (i, 0)),
        out_shape=jax.ShapeDtypeStruct((N1, K), jnp.int32),
    )(p1, p2t)

    idx_flat = jnp.pad(idx.reshape(N1 * K), (0, 16))

    interp_fn = pl.kernel(
        _interp_body,
        out_type=jax.ShapeDtypeStruct((N1, C), jnp.float32),
        mesh=plsc.VectorSubcoreMesh(core_axis_name="c", subcore_axis_name="s",
                                    num_cores=2, num_subcores=16),
        scratch_types=[
            pltpu.VMEM((QW * K + 16,), jnp.int32),
            pltpu.VMEM((QW * 16,), jnp.float32),
            pltpu.VMEM((N2 * 16,), jnp.float32),
            pltpu.VMEM((KCH, C), jnp.float32),
            pltpu.VMEM((KCH, C), jnp.float32),
            pltpu.VMEM((CH, C), jnp.float32),
            pltpu.VMEM((CH, C), jnp.float32),
            pltpu.SemaphoreType.DMA,
            pltpu.SemaphoreType.DMA,
            pltpu.SemaphoreType.DMA,
            pltpu.SemaphoreType.DMA,
        ],
    )
    interp = interp_fn(idx_flat, p1pad, p2pad, x2)

    y1, stats = pl.pallas_call(
        _lin1_body,
        grid=(NBLK,),
        in_specs=[
            pl.BlockSpec((BLK, 2 * C), lambda i: (i, 0)),
            pl.BlockSpec((2 * C, C), lambda i: (0, 0)),
            pl.BlockSpec((1, C), lambda i: (0, 0)),
        ],
        out_specs=[
            pl.BlockSpec((BLK, C), lambda i: (i, 0)),
            pl.BlockSpec((2, C), lambda i: (0, 0)),
        ],
        out_shape=[
            jax.ShapeDtypeStruct((N1, C), jnp.float32),
            jax.ShapeDtypeStruct((2, C), jnp.float32),
        ],
    )(x1, w1t, b1r)

    out = pl.pallas_call(
        _fin_body,
        grid=(N1 // BLK2,),
        in_specs=[
            pl.BlockSpec((BLK2, C), lambda i: (i, 0)),
            pl.BlockSpec((BLK2, C), lambda i: (i, 0)),
            pl.BlockSpec((2, C), lambda i: (0, 0)),
            pl.BlockSpec((1, C), lambda i: (0, 0)),
            pl.BlockSpec((1, C), lambda i: (0, 0)),
            pl.BlockSpec((C, C), lambda i: (0, 0)),
            pl.BlockSpec((1, C), lambda i: (0, 0)),
        ],
        out_specs=pl.BlockSpec((BLK2, C), lambda i: (i, 0)),
        out_shape=jax.ShapeDtypeStruct((N1, C), jnp.float32),
    )(y1, interp, stats, g1r, be1r, w2t, b2r)
    return out


# speedup vs baseline: 1.4177x; 1.4177x over previous
import functools

import jax
import jax.numpy as jnp
from jax import lax
from jax.experimental import pallas as pl
from jax.experimental.pallas import tpu as pltpu
from jax.experimental.pallas import tpu_sc as plsc

N1 = 16384
N2 = 4096
C = 512
K = 5
BLK = 256
NBLK = N1 // BLK
BLK2 = 2048
EPS = 1e-5
FINF = 3.0e38

NW = 32
H = N1 // 2
QW = H // NW
CH = 8
NCH = QW // CH
KCH = CH * K
NJ = C // 16


def _lin1_body(x1_ref, w1t_ref, b1_ref, y1_ref, stats_ref):
    i = pl.program_id(0)
    y1 = jnp.dot(x1_ref[...], w1t_ref[...],
                 preferred_element_type=jnp.float32) + b1_ref[...]
    y1_ref[...] = y1

    @pl.when(i == 0)
    def _():
        stats_ref[...] = jnp.zeros_like(stats_ref)

    s1 = jnp.sum(y1, axis=0)
    s2 = jnp.sum(y1 * y1, axis=0)
    stats_ref[...] += jnp.stack([s1, s2])


def _knn_body(p1_ref, p2t_ref, idx_ref):
    p1b = p1_ref[...]
    p2t = p2t_ref[...]
    rn1 = jnp.sum(p1b * p1b, axis=1, keepdims=True)
    rn2 = jnp.sum(p2t * p2t, axis=0, keepdims=True)
    pp = jnp.dot(p1b.astype(jnp.bfloat16), p2t.astype(jnp.bfloat16),
                 preferred_element_type=jnp.float32)
    d2 = rn1 - 2.0 * pp + rn2

    iota = lax.broadcasted_iota(jnp.int32, (BLK, N2), 1)
    work = d2
    cols = []
    for k in range(K):
        m = jnp.min(work, axis=1, keepdims=True)
        sel = work <= m
        idxk = jnp.min(jnp.where(sel, iota, N2), axis=1, keepdims=True)
        cols.append(idxk)
        if k < K - 1:
            work = jnp.where(iota == idxk, FINF, work)
    idx_ref[...] = jnp.concatenate(cols, axis=1)


def _interp_body(idx_hbm, p1pad_hbm, p2pad_hbm, x2_hbm, out_hbm,
                 idx_v, p1loc_v, p2loc_v, frows0_v, frows1_v,
                 out0_v, out1_v, semf0, semf1, semo0, semo1):
    cid = lax.axis_index("c")
    sid = lax.axis_index("s")
    wid = sid * 2 + cid
    qbase = wid * QW

    frows_b = (frows0_v, frows1_v)
    out_b = (out0_v, out1_v)
    semf_b = (semf0, semf1)
    semo_b = (semo0, semo1)

    pltpu.sync_copy(idx_hbm.at[pl.ds(qbase * K, QW * K + 16)], idx_v)
    pltpu.sync_copy(p1pad_hbm.at[pl.ds(qbase * 16, QW * 16)], p1loc_v)
    pltpu.sync_copy(p2pad_hbm, p2loc_v)

    def issue(ch, b):
        pltpu.async_copy(x2_hbm.at[idx_v.at[pl.ds(ch * KCH, KCH)]],
                         frows_b[b], semf_b[b])

    issue(0, 0)
    issue(1, 1)

    @pl.loop(0, NCH, step=2)
    def _chunk_pair(i):
        for b in range(2):
            ch = i + b
            qoff = ch * CH
            pltpu.make_async_copy(x2_hbm.at[idx_v.at[pl.ds(0, KCH)]],
                                  frows_b[b], semf_b[b]).wait()
            frows_v = frows_b[b]
            out_v = out_b[b]

            @plsc.parallel_loop(0, CH, unroll=2)
            def q_body(q):
                pq = p1loc_v[pl.ds((qoff + q) * 16, 16)]
                iv = idx_v[pl.ds((qoff + q) * K, 16)]
                wks = []
                for k in range(K):
                    ck = p2loc_v[pl.ds(iv[k] * 16, 16)]
                    d = ck - pq
                    sq = d * d
                    dk = sq[0] + sq[1] + sq[2]
                    wk = 1.0 / jnp.maximum(jnp.full((16,), dk, jnp.float32),
                                           1e-10)
                    wks.append(wk)
                winv = 1.0 / (wks[0] + wks[1] + wks[2] + wks[3] + wks[4])
                wn = [wk * winv for wk in wks]
                r = q * K
                for j in range(NJ):
                    sl = pl.ds(j * 16, 16)
                    acc = (wn[0] * frows_v[r, sl]
                           + wn[1] * frows_v[r + 1, sl]
                           + wn[2] * frows_v[r + 2, sl]
                           + wn[3] * frows_v[r + 3, sl]
                           + wn[4] * frows_v[r + 4, sl])
                    out_v[q, sl] = acc

            pltpu.sync_copy(out_v, out_hbm.at[pl.ds(qbase + qoff, CH), :])

            @pl.when(ch + 2 < NCH)
            def _():
                issue(ch + 2, b)


def _fin_body(y1_ref, interp_ref, stats_ref, g1_ref, be1_ref,
              w2t_ref, b2_ref, out_ref):
    h2 = jnp.dot(interp_ref[...], w2t_ref[...],
                 preferred_element_type=jnp.float32) + b2_ref[...]
    h2 = jnp.maximum(h2, 0.0)
    stats = stats_ref[...]
    mean = stats[0:1, :] * (1.0 / N1)
    var = stats[1:2, :] * (1.0 / N1) - mean * mean
    scale = g1_ref[...] * lax.rsqrt(var + EPS)
    h1 = jnp.maximum((y1_ref[...] - mean) * scale + be1_ref[...], 0.0)
    out_ref[...] = h1 + h2


def kernel(p1, x1, o1, p2, x2, o2, W1, b1, g1, be1, W2, b2):
    p2t = p2.T
    w1t = W1.T
    w2t = W2.T
    b1r = b1.reshape(1, C)
    b2r = b2.reshape(1, C)
    g1r = g1.reshape(1, C)
    be1r = be1.reshape(1, C)
    p1pad = jnp.pad(p1, ((0, 0), (0, 13))).reshape(N1 * 16)
    p2pad = jnp.pad(p2, ((0, 0), (0, 13))).reshape(N2 * 16)

    def knn_half(p1h):
        return pl.pallas_call(
            _knn_body,
            grid=(H // BLK,),
            in_specs=[
                pl.BlockSpec((BLK, 3), lambda i: (i, 0)),
                pl.BlockSpec((3, N2), lambda i: (0, 0)),
            ],
            out_specs=pl.BlockSpec((BLK, K), lambda i: (i, 0)),
            out_shape=jax.ShapeDtypeStruct((H, K), jnp.int32),
        )(p1h, p2t)

    interp_fn = pl.kernel(
        _interp_body,
        out_type=jax.ShapeDtypeStruct((H, C), jnp.float32),
        mesh=plsc.VectorSubcoreMesh(core_axis_name="c", subcore_axis_name="s",
                                    num_cores=2, num_subcores=16),
        scratch_types=[
            pltpu.VMEM((QW * K + 16,), jnp.int32),
            pltpu.VMEM((QW * 16,), jnp.float32),
            pltpu.VMEM((N2 * 16,), jnp.float32),
            pltpu.VMEM((KCH, C), jnp.float32),
            pltpu.VMEM((KCH, C), jnp.float32),
            pltpu.VMEM((CH, C), jnp.float32),
            pltpu.VMEM((CH, C), jnp.float32),
            pltpu.SemaphoreType.DMA,
            pltpu.SemaphoreType.DMA,
            pltpu.SemaphoreType.DMA,
            pltpu.SemaphoreType.DMA,
        ],
    )

    idx0 = knn_half(p1[:H])
    interp0 = interp_fn(jnp.pad(idx0.reshape(H * K), (0, 16)),
                        p1pad[:H * 16], p2pad, x2)
    idx1 = knn_half(p1[H:])
    interp1 = interp_fn(jnp.pad(idx1.reshape(H * K), (0, 16)),
                        p1pad[H * 16:], p2pad, x2)
    interp = jnp.concatenate([interp0, interp1], axis=0)

    y1, stats = pl.pallas_call(
        _lin1_body,
        grid=(NBLK,),
        in_specs=[
            pl.BlockSpec((BLK, 2 * C), lambda i: (i, 0)),
            pl.BlockSpec((2 * C, C), lambda i: (0, 0)),
            pl.BlockSpec((1, C), lambda i: (0, 0)),
        ],
        out_specs=[
            pl.BlockSpec((BLK, C), lambda i: (i, 0)),
            pl.BlockSpec((2, C), lambda i: (0, 0)),
        ],
        out_shape=[
            jax.ShapeDtypeStruct((N1, C), jnp.float32),
            jax.ShapeDtypeStruct((2, C), jnp.float32),
        ],
    )(x1, w1t, b1r)

    out = pl.pallas_call(
        _fin_body,
        grid=(N1 // BLK2,),
        in_specs=[
            pl.BlockSpec((BLK2, C), lambda i: (i, 0)),
            pl.BlockSpec((BLK2, C), lambda i: (i, 0)),
            pl.BlockSpec((2, C), lambda i: (0, 0)),
            pl.BlockSpec((1, C), lambda i: (0, 0)),
            pl.BlockSpec((1, C), lambda i: (0, 0)),
            pl.BlockSpec((C, C), lambda i: (0, 0)),
            pl.BlockSpec((1, C), lambda i: (0, 0)),
        ],
        out_specs=pl.BlockSpec((BLK2, C), lambda i: (i, 0)),
        out_shape=jax.ShapeDtypeStruct((N1, C), jnp.float32),
    )(y1, interp, stats, g1r, be1r, w2t, b2r)
    return out


# speedup vs baseline: 1.4244x; 1.0047x over previous
import functools

import jax
import jax.numpy as jnp
from jax import lax
from jax.experimental import pallas as pl
from jax.experimental.pallas import tpu as pltpu
from jax.experimental.pallas import tpu_sc as plsc

N1 = 16384
N2 = 4096
C = 512
K = 5
BLK = 256
NBLK = N1 // BLK
BLK2 = 2048
EPS = 1e-5
FINF = 3.0e38

NW = 32
H = N1 // 4
QW = H // NW
CH = 8
NCH = QW // CH
KCH = CH * K
NJ = C // 16


def _lin1_body(x1_ref, w1t_ref, b1_ref, y1_ref, stats_ref):
    i = pl.program_id(0)
    y1 = jnp.dot(x1_ref[...], w1t_ref[...],
                 preferred_element_type=jnp.float32) + b1_ref[...]
    y1_ref[...] = y1

    @pl.when(i == 0)
    def _():
        stats_ref[...] = jnp.zeros_like(stats_ref)

    s1 = jnp.sum(y1, axis=0)
    s2 = jnp.sum(y1 * y1, axis=0)
    stats_ref[...] += jnp.stack([s1, s2])


def _knn_body(p1_ref, p2t_ref, idx_ref):
    p1b = p1_ref[...]
    p2t = p2t_ref[...]
    rn1 = jnp.sum(p1b * p1b, axis=1, keepdims=True)
    rn2 = jnp.sum(p2t * p2t, axis=0, keepdims=True)
    pp = jnp.dot(p1b.astype(jnp.bfloat16), p2t.astype(jnp.bfloat16),
                 preferred_element_type=jnp.float32)
    d2 = rn1 - 2.0 * pp + rn2

    iota = lax.broadcasted_iota(jnp.int32, (BLK, N2), 1)
    work = d2
    cols = []
    for k in range(K):
        m = jnp.min(work, axis=1, keepdims=True)
        sel = work <= m
        idxk = jnp.min(jnp.where(sel, iota, N2), axis=1, keepdims=True)
        cols.append(idxk)
        if k < K - 1:
            work = jnp.where(iota == idxk, FINF, work)
    idx_ref[...] = jnp.concatenate(cols, axis=1)


def _interp_body(idx_hbm, p1pad_hbm, p2pad_hbm, x2_hbm, out_hbm,
                 idx_v, p1loc_v, p2loc_v, frows0_v, frows1_v,
                 out0_v, out1_v, semf0, semf1, semo0, semo1):
    cid = lax.axis_index("c")
    sid = lax.axis_index("s")
    wid = sid * 2 + cid
    qbase = wid * QW

    frows_b = (frows0_v, frows1_v)
    out_b = (out0_v, out1_v)
    semf_b = (semf0, semf1)
    semo_b = (semo0, semo1)

    pltpu.sync_copy(idx_hbm.at[pl.ds(qbase * K, QW * K + 16)], idx_v)
    pltpu.sync_copy(p1pad_hbm.at[pl.ds(qbase * 16, QW * 16)], p1loc_v)
    pltpu.sync_copy(p2pad_hbm, p2loc_v)

    def issue(ch, b):
        pltpu.async_copy(x2_hbm.at[idx_v.at[pl.ds(ch * KCH, KCH)]],
                         frows_b[b], semf_b[b])

    issue(0, 0)
    issue(1, 1)

    @pl.loop(0, NCH, step=2)
    def _chunk_pair(i):
        for b in range(2):
            ch = i + b
            qoff = ch * CH
            pltpu.make_async_copy(x2_hbm.at[idx_v.at[pl.ds(0, KCH)]],
                                  frows_b[b], semf_b[b]).wait()
            frows_v = frows_b[b]
            out_v = out_b[b]

            @plsc.parallel_loop(0, CH, unroll=2)
            def q_body(q):
                pq = p1loc_v[pl.ds((qoff + q) * 16, 16)]
                iv = idx_v[pl.ds((qoff + q) * K, 16)]
                wks = []
                for k in range(K):
                    ck = p2loc_v[pl.ds(iv[k] * 16, 16)]
                    d = ck - pq
                    sq = d * d
                    dk = sq[0] + sq[1] + sq[2]
                    wk = 1.0 / jnp.maximum(jnp.full((16,), dk, jnp.float32),
                                           1e-10)
                    wks.append(wk)
                winv = 1.0 / (wks[0] + wks[1] + wks[2] + wks[3] + wks[4])
                wn = [wk * winv for wk in wks]
                r = q * K
                for j in range(NJ):
                    sl = pl.ds(j * 16, 16)
                    acc = (wn[0] * frows_v[r, sl]
                           + wn[1] * frows_v[r + 1, sl]
                           + wn[2] * frows_v[r + 2, sl]
                           + wn[3] * frows_v[r + 3, sl]
                           + wn[4] * frows_v[r + 4, sl])
                    out_v[q, sl] = acc

            pltpu.sync_copy(out_v, out_hbm.at[pl.ds(qbase + qoff, CH), :])

            @pl.when(ch + 2 < NCH)
            def _():
                issue(ch + 2, b)


def _fin_body(y1_ref, interp_ref, stats_ref, g1_ref, be1_ref,
              w2t_ref, b2_ref, out_ref):
    h2 = jnp.dot(interp_ref[...], w2t_ref[...],
                 preferred_element_type=jnp.float32) + b2_ref[...]
    h2 = jnp.maximum(h2, 0.0)
    stats = stats_ref[...]
    mean = stats[0:1, :] * (1.0 / N1)
    var = stats[1:2, :] * (1.0 / N1) - mean * mean
    scale = g1_ref[...] * lax.rsqrt(var + EPS)
    h1 = jnp.maximum((y1_ref[...] - mean) * scale + be1_ref[...], 0.0)
    out_ref[...] = h1 + h2


def kernel(p1, x1, o1, p2, x2, o2, W1, b1, g1, be1, W2, b2):
    p2t = p2.T
    w1t = W1.T
    w2t = W2.T
    b1r = b1.reshape(1, C)
    b2r = b2.reshape(1, C)
    g1r = g1.reshape(1, C)
    be1r = be1.reshape(1, C)
    p1pad = jnp.pad(p1, ((0, 0), (0, 13))).reshape(N1 * 16)
    p2pad = jnp.pad(p2, ((0, 0), (0, 13))).reshape(N2 * 16)

    def knn_half(p1h):
        return pl.pallas_call(
            _knn_body,
            grid=(H // BLK,),
            in_specs=[
                pl.BlockSpec((BLK, 3), lambda i: (i, 0)),
                pl.BlockSpec((3, N2), lambda i: (0, 0)),
            ],
            out_specs=pl.BlockSpec((BLK, K), lambda i: (i, 0)),
            out_shape=jax.ShapeDtypeStruct((H, K), jnp.int32),
        )(p1h, p2t)

    interp_fn = pl.kernel(
        _interp_body,
        out_type=jax.ShapeDtypeStruct((H, C), jnp.float32),
        mesh=plsc.VectorSubcoreMesh(core_axis_name="c", subcore_axis_name="s",
                                    num_cores=2, num_subcores=16),
        scratch_types=[
            pltpu.VMEM((QW * K + 16,), jnp.int32),
            pltpu.VMEM((QW * 16,), jnp.float32),
            pltpu.VMEM((N2 * 16,), jnp.float32),
            pltpu.VMEM((KCH, C), jnp.float32),
            pltpu.VMEM((KCH, C), jnp.float32),
            pltpu.VMEM((CH, C), jnp.float32),
            pltpu.VMEM((CH, C), jnp.float32),
            pltpu.SemaphoreType.DMA,
            pltpu.SemaphoreType.DMA,
            pltpu.SemaphoreType.DMA,
            pltpu.SemaphoreType.DMA,
        ],
    )

    interps = []
    for hh in range(N1 // H):
        idxh = knn_half(p1[hh * H:(hh + 1) * H])
        interps.append(interp_fn(jnp.pad(idxh.reshape(H * K), (0, 16)),
                                 p1pad[hh * H * 16:(hh + 1) * H * 16],
                                 p2pad, x2))
    interp = jnp.concatenate(interps, axis=0)

    y1, stats = pl.pallas_call(
        _lin1_body,
        grid=(NBLK,),
        in_specs=[
            pl.BlockSpec((BLK, 2 * C), lambda i: (i, 0)),
            pl.BlockSpec((2 * C, C), lambda i: (0, 0)),
            pl.BlockSpec((1, C), lambda i: (0, 0)),
        ],
        out_specs=[
            pl.BlockSpec((BLK, C), lambda i: (i, 0)),
            pl.BlockSpec((2, C), lambda i: (0, 0)),
        ],
        out_shape=[
            jax.ShapeDtypeStruct((N1, C), jnp.float32),
            jax.ShapeDtypeStruct((2, C), jnp.float32),
        ],
    )(x1, w1t, b1r)

    out = pl.pallas_call(
        _fin_body,
        grid=(N1 // BLK2,),
        in_specs=[
            pl.BlockSpec((BLK2, C), lambda i: (i, 0)),
            pl.BlockSpec((BLK2, C), lambda i: (i, 0)),
            pl.BlockSpec((2, C), lambda i: (0, 0)),
            pl.BlockSpec((1, C), lambda i: (0, 0)),
            pl.BlockSpec((1, C), lambda i: (0, 0)),
            pl.BlockSpec((C, C), lambda i: (0, 0)),
            pl.BlockSpec((1, C), lambda i: (0, 0)),
        ],
        out_specs=pl.BlockSpec((BLK2, C), lambda i: (i, 0)),
        out_shape=jax.ShapeDtypeStruct((N1, C), jnp.float32),
    )(y1, interp, stats, g1r, be1r, w2t, b2r)
    return out


# speedup vs baseline: 1.6355x; 1.1482x over previous
import functools

import jax
import jax.numpy as jnp
from jax import lax
from jax.experimental import pallas as pl
from jax.experimental.pallas import tpu as pltpu
from jax.experimental.pallas import tpu_sc as plsc

N1 = 16384
N2 = 4096
C = 512
K = 5
BLK = 256
NBLK = N1 // BLK
BLK2 = 2048
EPS = 1e-5
FINF = 3.0e38

NW = 32
H = N1 // 4
QW = H // NW
CH = 8
NCH = QW // CH
KCH = CH * K
NJ = C // 16


def _lin1_body(x1_ref, w1t_ref, b1_ref, y1_ref, stats_ref):
    i = pl.program_id(0)
    y1 = jnp.dot(x1_ref[...], w1t_ref[...],
                 preferred_element_type=jnp.float32) + b1_ref[...]
    y1_ref[...] = y1

    @pl.when(i == 0)
    def _():
        stats_ref[...] = jnp.zeros_like(stats_ref)

    s1 = jnp.sum(y1, axis=0)
    s2 = jnp.sum(y1 * y1, axis=0)
    stats_ref[...] += jnp.stack([s1, s2])


def _knn_body(p1_ref, p2t_ref, idx_ref):
    p1b = p1_ref[...]
    p2t = p2t_ref[...]
    rn1 = jnp.sum(p1b * p1b, axis=1, keepdims=True)
    rn2 = jnp.sum(p2t * p2t, axis=0, keepdims=True)
    pp = jnp.dot(p1b.astype(jnp.bfloat16), p2t.astype(jnp.bfloat16),
                 preferred_element_type=jnp.float32)
    d2 = rn1 - 2.0 * pp + rn2

    iota = lax.broadcasted_iota(jnp.int32, (BLK, N2), 1).astype(jnp.float32)
    work = d2
    cols = []
    for k in range(K):
        m = jnp.min(work, axis=1, keepdims=True)
        sel = work <= m
        idxk = jnp.min(jnp.where(sel, iota, FINF), axis=1, keepdims=True)
        cols.append(idxk)
        if k < K - 1:
            work = jnp.where(iota == idxk, FINF, work)
    idx_ref[...] = jnp.concatenate(cols, axis=1).astype(jnp.int32)


def _interp_body(idx_hbm, p1pad_hbm, p2pad_hbm, x2_hbm, out_hbm,
                 idx_v, p1loc_v, p2loc_v, frows0_v, frows1_v,
                 out0_v, out1_v, semf0, semf1, semo0, semo1):
    cid = lax.axis_index("c")
    sid = lax.axis_index("s")
    wid = sid * 2 + cid
    qbase = wid * QW

    frows_b = (frows0_v, frows1_v)
    out_b = (out0_v, out1_v)
    semf_b = (semf0, semf1)
    semo_b = (semo0, semo1)

    pltpu.sync_copy(idx_hbm.at[pl.ds(qbase * K, QW * K + 16)], idx_v)
    pltpu.sync_copy(p1pad_hbm.at[pl.ds(qbase * 16, QW * 16)], p1loc_v)
    pltpu.sync_copy(p2pad_hbm, p2loc_v)

    def issue(ch, b):
        pltpu.async_copy(x2_hbm.at[idx_v.at[pl.ds(ch * KCH, KCH)]],
                         frows_b[b], semf_b[b])

    issue(0, 0)
    issue(1, 1)

    @pl.loop(0, NCH, step=2)
    def _chunk_pair(i):
        for b in range(2):
            ch = i + b
            qoff = ch * CH
            pltpu.make_async_copy(x2_hbm.at[idx_v.at[pl.ds(0, KCH)]],
                                  frows_b[b], semf_b[b]).wait()
            frows_v = frows_b[b]
            out_v = out_b[b]

            @plsc.parallel_loop(0, CH, unroll=2)
            def q_body(q):
                pq = p1loc_v[pl.ds((qoff + q) * 16, 16)]
                iv = idx_v[pl.ds((qoff + q) * K, 16)]
                wks = []
                for k in range(K):
                    ck = p2loc_v[pl.ds(iv[k] * 16, 16)]
                    d = ck - pq
                    sq = d * d
                    dk = sq[0] + sq[1] + sq[2]
                    wk = 1.0 / jnp.maximum(jnp.full((16,), dk, jnp.float32),
                                           1e-10)
                    wks.append(wk)
                winv = 1.0 / (wks[0] + wks[1] + wks[2] + wks[3] + wks[4])
                wn = [wk * winv for wk in wks]
                r = q * K
                for j in range(NJ):
                    sl = pl.ds(j * 16, 16)
                    acc = (wn[0] * frows_v[r, sl]
                           + wn[1] * frows_v[r + 1, sl]
                           + wn[2] * frows_v[r + 2, sl]
                           + wn[3] * frows_v[r + 3, sl]
                           + wn[4] * frows_v[r + 4, sl])
                    out_v[q, sl] = acc

            pltpu.sync_copy(out_v, out_hbm.at[pl.ds(qbase + qoff, CH), :])

            @pl.when(ch + 2 < NCH)
            def _():
                issue(ch + 2, b)


def _fin_body(y1_ref, interp_ref, stats_ref, g1_ref, be1_ref,
              w2t_ref, b2_ref, out_ref):
    h2 = jnp.dot(interp_ref[...], w2t_ref[...],
                 preferred_element_type=jnp.float32) + b2_ref[...]
    h2 = jnp.maximum(h2, 0.0)
    stats = stats_ref[...]
    mean = stats[0:1, :] * (1.0 / N1)
    var = stats[1:2, :] * (1.0 / N1) - mean * mean
    scale = g1_ref[...] * lax.rsqrt(var + EPS)
    h1 = jnp.maximum((y1_ref[...] - mean) * scale + be1_ref[...], 0.0)
    out_ref[...] = h1 + h2


def kernel(p1, x1, o1, p2, x2, o2, W1, b1, g1, be1, W2, b2):
    p2t = p2.T
    w1t = W1.T
    w2t = W2.T
    b1r = b1.reshape(1, C)
    b2r = b2.reshape(1, C)
    g1r = g1.reshape(1, C)
    be1r = be1.reshape(1, C)
    p1pad = jnp.pad(p1, ((0, 0), (0, 13))).reshape(N1 * 16)
    p2pad = jnp.pad(p2, ((0, 0), (0, 13))).reshape(N2 * 16)

    def knn_half(p1h):
        return pl.pallas_call(
            _knn_body,
            grid=(H // BLK,),
            in_specs=[
                pl.BlockSpec((BLK, 3), lambda i: (i, 0)),
                pl.BlockSpec((3, N2), lambda i: (0, 0)),
            ],
            out_specs=pl.BlockSpec((BLK, K), lambda i: (i, 0)),
            out_shape=jax.ShapeDtypeStruct((H, K), jnp.int32),
        )(p1h, p2t)

    interp_fn = pl.kernel(
        _interp_body,
        out_type=jax.ShapeDtypeStruct((H, C), jnp.float32),
        mesh=plsc.VectorSubcoreMesh(core_axis_name="c", subcore_axis_name="s",
                                    num_cores=2, num_subcores=16),
        scratch_types=[
            pltpu.VMEM((QW * K + 16,), jnp.int32),
            pltpu.VMEM((QW * 16,), jnp.float32),
            pltpu.VMEM((N2 * 16,), jnp.float32),
            pltpu.VMEM((KCH, C), jnp.float32),
            pltpu.VMEM((KCH, C), jnp.float32),
            pltpu.VMEM((CH, C), jnp.float32),
            pltpu.VMEM((CH, C), jnp.float32),
            pltpu.SemaphoreType.DMA,
            pltpu.SemaphoreType.DMA,
            pltpu.SemaphoreType.DMA,
            pltpu.SemaphoreType.DMA,
        ],
    )

    interps = []
    for hh in range(N1 // H):
        idxh = knn_half(p1[hh * H:(hh + 1) * H])
        interps.append(interp_fn(jnp.pad(idxh.reshape(H * K), (0, 16)),
                                 p1pad[hh * H * 16:(hh + 1) * H * 16],
                                 p2pad, x2))
    interp = jnp.concatenate(interps, axis=0)

    y1, stats = pl.pallas_call(
        _lin1_body,
        grid=(NBLK,),
        in_specs=[
            pl.BlockSpec((BLK, 2 * C), lambda i: (i, 0)),
            pl.BlockSpec((2 * C, C), lambda i: (0, 0)),
            pl.BlockSpec((1, C), lambda i: (0, 0)),
        ],
        out_specs=[
            pl.BlockSpec((BLK, C), lambda i: (i, 0)),
            pl.BlockSpec((2, C), lambda i: (0, 0)),
        ],
        out_shape=[
            jax.ShapeDtypeStruct((N1, C), jnp.float32),
            jax.ShapeDtypeStruct((2, C), jnp.float32),
        ],
    )(x1, w1t, b1r)

    out = pl.pallas_call(
        _fin_body,
        grid=(N1 // BLK2,),
        in_specs=[
            pl.BlockSpec((BLK2, C), lambda i: (i, 0)),
            pl.BlockSpec((BLK2, C), lambda i: (i, 0)),
            pl.BlockSpec((2, C), lambda i: (0, 0)),
            pl.BlockSpec((1, C), lambda i: (0, 0)),
            pl.BlockSpec((1, C), lambda i: (0, 0)),
            pl.BlockSpec((C, C), lambda i: (0, 0)),
            pl.BlockSpec((1, C), lambda i: (0, 0)),
        ],
        out_specs=pl.BlockSpec((BLK2, C), lambda i: (i, 0)),
        out_shape=jax.ShapeDtypeStruct((N1, C), jnp.float32),
    )(y1, interp, stats, g1r, be1r, w2t, b2r)
    return out


# speedup vs baseline: 1.6773x; 1.0255x over previous
import functools

import jax
import jax.numpy as jnp
from jax import lax
from jax.experimental import pallas as pl
from jax.experimental.pallas import tpu as pltpu
from jax.experimental.pallas import tpu_sc as plsc

N1 = 16384
N2 = 4096
C = 512
K = 5
BLK = 512
NBLK = N1 // BLK
BLK2 = 2048
EPS = 1e-5
FINF = 3.0e38

NW = 32
H = N1 // 4
QW = H // NW
CH = 8
NCH = QW // CH
KCH = CH * K
NJ = C // 16


def _lin1_body(x1_ref, w1t_ref, b1_ref, y1_ref, stats_ref):
    i = pl.program_id(0)
    y1 = jnp.dot(x1_ref[...], w1t_ref[...],
                 preferred_element_type=jnp.float32) + b1_ref[...]
    y1_ref[...] = y1

    @pl.when(i == 0)
    def _():
        stats_ref[...] = jnp.zeros_like(stats_ref)

    s1 = jnp.sum(y1, axis=0)
    s2 = jnp.sum(y1 * y1, axis=0)
    stats_ref[...] += jnp.stack([s1, s2])


def _knn_body(p1_ref, p2t_ref, idx_ref):
    p1b = p1_ref[...]
    p2t = p2t_ref[...]
    rn1 = jnp.sum(p1b * p1b, axis=1, keepdims=True)
    rn2 = jnp.sum(p2t * p2t, axis=0, keepdims=True)
    pp = jnp.dot(p1b.astype(jnp.bfloat16), p2t.astype(jnp.bfloat16),
                 preferred_element_type=jnp.float32)
    d2 = rn1 - 2.0 * pp + rn2

    iota = lax.broadcasted_iota(jnp.int32, (BLK, N2), 1).astype(jnp.float32)
    work = d2
    cols = []
    for k in range(K):
        m = jnp.min(work, axis=1, keepdims=True)
        sel = work <= m
        idxk = jnp.min(jnp.where(sel, iota, FINF), axis=1, keepdims=True)
        cols.append(idxk)
        if k < K - 1:
            work = jnp.where(iota == idxk, FINF, work)
    idx_ref[...] = jnp.concatenate(cols, axis=1).astype(jnp.int32)


def _interp_body(idx_hbm, p1pad_hbm, p2pad_hbm, x2_hbm, out_hbm,
                 idx_v, p1loc_v, p2loc_v, frows0_v, frows1_v,
                 out0_v, out1_v, semf0, semf1, semo0, semo1):
    cid = lax.axis_index("c")
    sid = lax.axis_index("s")
    wid = sid * 2 + cid
    qbase = wid * QW

    frows_b = (frows0_v, frows1_v)
    out_b = (out0_v, out1_v)
    semf_b = (semf0, semf1)
    semo_b = (semo0, semo1)

    pltpu.sync_copy(idx_hbm.at[pl.ds(qbase * K, QW * K + 16)], idx_v)
    pltpu.sync_copy(p1pad_hbm.at[pl.ds(qbase * 16, QW * 16)], p1loc_v)
    pltpu.sync_copy(p2pad_hbm, p2loc_v)

    def issue(ch, b):
        pltpu.async_copy(x2_hbm.at[idx_v.at[pl.ds(ch * KCH, KCH)]],
                         frows_b[b], semf_b[b])

    issue(0, 0)
    issue(1, 1)

    @pl.loop(0, NCH, step=2)
    def _chunk_pair(i):
        for b in range(2):
            ch = i + b
            qoff = ch * CH
            pltpu.make_async_copy(x2_hbm.at[idx_v.at[pl.ds(0, KCH)]],
                                  frows_b[b], semf_b[b]).wait()
            frows_v = frows_b[b]
            out_v = out_b[b]

            @plsc.parallel_loop(0, CH, unroll=2)
            def q_body(q):
                pq = p1loc_v[pl.ds((qoff + q) * 16, 16)]
                iv = idx_v[pl.ds((qoff + q) * K, 16)]
                wks = []
                for k in range(K):
                    ck = p2loc_v[pl.ds(iv[k] * 16, 16)]
                    d = ck - pq
                    sq = d * d
                    dk = sq[0] + sq[1] + sq[2]
                    wk = 1.0 / jnp.maximum(jnp.full((16,), dk, jnp.float32),
                                           1e-10)
                    wks.append(wk)
                winv = 1.0 / (wks[0] + wks[1] + wks[2] + wks[3] + wks[4])
                wn = [wk * winv for wk in wks]
                r = q * K
                for j in range(NJ):
                    sl = pl.ds(j * 16, 16)
                    acc = (wn[0] * frows_v[r, sl]
                           + wn[1] * frows_v[r + 1, sl]
                           + wn[2] * frows_v[r + 2, sl]
                           + wn[3] * frows_v[r + 3, sl]
                           + wn[4] * frows_v[r + 4, sl])
                    out_v[q, sl] = acc

            pltpu.sync_copy(out_v, out_hbm.at[pl.ds(qbase + qoff, CH), :])

            @pl.when(ch + 2 < NCH)
            def _():
                issue(ch + 2, b)


def _fin_body(y1_ref, interp_ref, stats_ref, g1_ref, be1_ref,
              w2t_ref, b2_ref, out_ref):
    h2 = jnp.dot(interp_ref[...], w2t_ref[...],
                 preferred_element_type=jnp.float32) + b2_ref[...]
    h2 = jnp.maximum(h2, 0.0)
    stats = stats_ref[...]
    mean = stats[0:1, :] * (1.0 / N1)
    var = stats[1:2, :] * (1.0 / N1) - mean * mean
    scale = g1_ref[...] * lax.rsqrt(var + EPS)
    h1 = jnp.maximum((y1_ref[...] - mean) * scale + be1_ref[...], 0.0)
    out_ref[...] = h1 + h2


def kernel(p1, x1, o1, p2, x2, o2, W1, b1, g1, be1, W2, b2):
    p2t = p2.T
    w1t = W1.T
    w2t = W2.T
    b1r = b1.reshape(1, C)
    b2r = b2.reshape(1, C)
    g1r = g1.reshape(1, C)
    be1r = be1.reshape(1, C)
    p1pad = jnp.pad(p1, ((0, 0), (0, 13))).reshape(N1 * 16)
    p2pad = jnp.pad(p2, ((0, 0), (0, 13))).reshape(N2 * 16)

    def knn_half(p1h):
        return pl.pallas_call(
            _knn_body,
            grid=(H // BLK,),
            in_specs=[
                pl.BlockSpec((BLK, 3), lambda i: (i, 0)),
                pl.BlockSpec((3, N2), lambda i: (0, 0)),
            ],
            out_specs=pl.BlockSpec((BLK, K), lambda i: (i, 0)),
            out_shape=jax.ShapeDtypeStruct((H, K), jnp.int32),
        )(p1h, p2t)

    interp_fn = pl.kernel(
        _interp_body,
        out_type=jax.ShapeDtypeStruct((H, C), jnp.float32),
        mesh=plsc.VectorSubcoreMesh(core_axis_name="c", subcore_axis_name="s",
                                    num_cores=2, num_subcores=16),
        scratch_types=[
            pltpu.VMEM((QW * K + 16,), jnp.int32),
            pltpu.VMEM((QW * 16,), jnp.float32),
            pltpu.VMEM((N2 * 16,), jnp.float32),
            pltpu.VMEM((KCH, C), jnp.float32),
            pltpu.VMEM((KCH, C), jnp.float32),
            pltpu.VMEM((CH, C), jnp.float32),
            pltpu.VMEM((CH, C), jnp.float32),
            pltpu.SemaphoreType.DMA,
            pltpu.SemaphoreType.DMA,
            pltpu.SemaphoreType.DMA,
            pltpu.SemaphoreType.DMA,
        ],
    )

    interps = []
    for hh in range(N1 // H):
        idxh = knn_half(p1[hh * H:(hh + 1) * H])
        interps.append(interp_fn(jnp.pad(idxh.reshape(H * K), (0, 16)),
                                 p1pad[hh * H * 16:(hh + 1) * H * 16],
                                 p2pad, x2))
    interp = jnp.concatenate(interps, axis=0)

    y1, stats = pl.pallas_call(
        _lin1_body,
        grid=(NBLK,),
        in_specs=[
            pl.BlockSpec((BLK, 2 * C), lambda i: (i, 0)),
            pl.BlockSpec((2 * C, C), lambda i: (0, 0)),
            pl.BlockSpec((1, C), lambda i: (0, 0)),
        ],
        out_specs=[
            pl.BlockSpec((BLK, C), lambda i: (i, 0)),
            pl.BlockSpec((2, C), lambda i: (0, 0)),
        ],
        out_shape=[
            jax.ShapeDtypeStruct((N1, C), jnp.float32),
            jax.ShapeDtypeStruct((2, C), jnp.float32),
        ],
    )(x1, w1t, b1r)

    out = pl.pallas_call(
        _fin_body,
        grid=(N1 // BLK2,),
        in_specs=[
            pl.BlockSpec((BLK2, C), lambda i: (i, 0)),
            pl.BlockSpec((BLK2, C), lambda i: (i, 0)),
            pl.BlockSpec((2, C), lambda i: (0, 0)),
            pl.BlockSpec((1, C), lambda i: (0, 0)),
            pl.BlockSpec((1, C), lambda i: (0, 0)),
            pl.BlockSpec((C, C), lambda i: (0, 0)),
            pl.BlockSpec((1, C), lambda i: (0, 0)),
        ],
        out_specs=pl.BlockSpec((BLK2, C), lambda i: (i, 0)),
        out_shape=jax.ShapeDtypeStruct((N1, C), jnp.float32),
    )(y1, interp, stats, g1r, be1r, w2t, b2r)
    return out
